# baseline (speedup 1.0000x reference)
import jax
import jax.numpy as jnp
from jax import lax
from jax.experimental import pallas as pl
from jax.experimental.pallas import tpu as pltpu
from jax.experimental.pallas import tpu_sc as plsc

_N = 10000
_E = 320000
_R = 3
_G = 64
_EMB = 128
_HID = 256
_NCLS = 10
_SLAB = 10112

_NS = 16
_K = 125
_NB = _E // _NS // _K
_NBH = _NB // 2
_CH = 40
_STR = _SLAB // _NS

_BN = 1000
_NBLK = _N // _BN


def _make_sc_scatter(nb: int):
    nc = nb // _CH
    mesh = plsc.VectorSubcoreMesh(core_axis_name="c", subcore_axis_name="s")
    out_type = jax.ShapeDtypeStruct((2, _SLAB, 128), jnp.float32)
    scratch = [
        pltpu.VMEM((_CH, _K), jnp.int32),
        pltpu.VMEM((_CH, _K), jnp.int32),
        pltpu.VMEM((2, _K, 128), jnp.float32),
        pltpu.VMEM_SHARED((_SLAB, 128), jnp.float32),
        pltpu.SemaphoreType.DMA,
        pltpu.SemaphoreType.DMA,
    ]

    def body(tbl0, tbl1, seg0_hbm, seg1_hbm, src0_hbm, src1_hbm, za,
             a_out, src_v, seg_v, rows_v, a_sh, sem0, sem1):
        cid = lax.axis_index("c")
        sid = lax.axis_index("s")

        pltpu.sync_copy(za, a_sh.at[pl.ds(sid * _STR, _STR)])

        plsc.subcore_barrier()

        def edge_pass(tbl, seg_hbm, src_hbm):
            def chunk(ci, carry):
                pltpu.sync_copy(seg_hbm.at[sid, pl.ds(ci * _CH, _CH)], seg_v)
                pltpu.sync_copy(src_hbm.at[sid, pl.ds(ci * _CH, _CH)], src_v)
                pltpu.async_copy(tbl.at[src_v.at[0]], rows_v.at[0], sem0)

                def pair(jp, c2):
                    j0 = 2 * jp
                    pltpu.make_async_copy(tbl.at[src_v.at[j0]],
                                          rows_v.at[0], sem0).wait()
                    pltpu.async_copy(tbl.at[src_v.at[j0 + 1]],
                                     rows_v.at[1], sem1)
                    pltpu.sync_copy(rows_v.at[0], a_sh.at[seg_v.at[j0]],
                                    add=True)
                    pltpu.make_async_copy(tbl.at[src_v.at[j0 + 1]],
                                          rows_v.at[1], sem1).wait()

                    @pl.when(jp < _CH // 2 - 1)
                    def _():
                        pltpu.async_copy(tbl.at[src_v.at[j0 + 2]],
                                         rows_v.at[0], sem0)

                    pltpu.sync_copy(rows_v.at[1], a_sh.at[seg_v.at[j0 + 1]],
                                    add=True)
                    return c2
                lax.fori_loop(0, _CH // 2, pair, 0)
                return carry
            lax.fori_loop(0, nc, chunk, 0)

        @pl.when(cid == 0)
        def _():
            edge_pass(tbl0, seg0_hbm, src0_hbm)

        @pl.when(cid == 1)
        def _():
            edge_pass(tbl1, seg1_hbm, src1_hbm)

        plsc.subcore_barrier()

        pltpu.sync_copy(a_sh.at[pl.ds(sid * _STR, _STR)],
                        a_out.at[cid, pl.ds(sid * _STR, _STR)])

    return pl.kernel(body, out_type=out_type, mesh=mesh,
                     scratch_types=scratch)


def _embed_body(s_ref, c_ref, se_ref, ce_ref, out_ref):
    sval = s_ref[0]
    cval = c_ref[0]
    ohs = (lax.broadcasted_iota(jnp.int32, (16, _BN), 0) == sval
           ).astype(jnp.float32)
    ohc = (lax.broadcasted_iota(jnp.int32, (16, _BN), 0) == cval
           ).astype(jnp.float32)
    dn = (((0,), (0,)), ((), ()))
    out_ref[...] = (
        lax.dot_general(ohs, se_ref[...], dn,
                        preferred_element_type=jnp.float32)
        + lax.dot_general(ohc, ce_ref[...], dn,
                          preferred_element_type=jnp.float32))


def _embed(s3, c3, se, ce):
    return pl.pallas_call(
        _embed_body,
        grid=(_NBLK,),
        in_specs=[
            pl.BlockSpec((1, 1, _BN), lambda i: (i, 0, 0)),
            pl.BlockSpec((1, 1, _BN), lambda i: (i, 0, 0)),
            pl.BlockSpec((16, _EMB), lambda i: (0, 0)),
            pl.BlockSpec((16, _EMB), lambda i: (0, 0)),
        ],
        out_specs=pl.BlockSpec((_BN, _EMB), lambda i: (i, 0)),
        out_shape=jax.ShapeDtypeStruct((_N, _EMB), jnp.float32),
    )(s3, c3, se, ce)


def _layer1_body(h_ref, s01_ref, s22_ref, cnt_ref, w_ref, b_ref,
                 out0_ref, out1_ref):
    inv = 1.0 / jnp.maximum(cnt_ref[...], 1.0)
    x = jnp.concatenate([
        h_ref[...],
        s01_ref[0] * inv[:, 0:1],
        s01_ref[1] * inv[:, 1:2],
        s22_ref[0] * inv[:, 2:3],
        s22_ref[1] * inv[:, 3:4],
    ], axis=1)
    acc = lax.dot_general(x, w_ref[...], (((1,), (0,)), ((), ())),
                          preferred_element_type=jnp.float32)
    acc = jnp.maximum(acc + b_ref[...], 0.0)
    out0_ref[...] = acc[:, :128]
    out1_ref[...] = acc[:, 128:]


def _layer1(h, s01, s22, cntp, wc, b):
    return pl.pallas_call(
        _layer1_body,
        grid=(_NBLK,),
        in_specs=[
            pl.BlockSpec((_BN, _EMB), lambda i: (i, 0)),
            pl.BlockSpec((2, _BN, 128), lambda i: (0, i, 0)),
            pl.BlockSpec((2, _BN, 128), lambda i: (0, i, 0)),
            pl.BlockSpec((_BN, 8), lambda i: (i, 0)),
            pl.BlockSpec((5 * _EMB, _HID), lambda i: (0, 0)),
            pl.BlockSpec((1, _HID), lambda i: (0, 0)),
        ],
        out_specs=[pl.BlockSpec((_BN, 128), lambda i: (i, 0)),
                   pl.BlockSpec((_BN, 128), lambda i: (i, 0))],
        out_shape=[jax.ShapeDtypeStruct((_N, 128), jnp.float32),
                   jax.ShapeDtypeStruct((_N, 128), jnp.float32)],
    )(h, s01, s22, cntp, wc, b)


def _layer2_pool_body(h0_ref, h1_ref, t01_ref, t20_ref, t12_ref, cnt_ref,
                      w_ref, b_ref, batch_ref, lw_ref, lb_ref, out_ref,
                      psum, gcnt):
    i = pl.program_id(0)

    @pl.when(i == 0)
    def _():
        psum[...] = jnp.zeros_like(psum)
        gcnt[...] = jnp.zeros_like(gcnt)

    inv = 1.0 / jnp.maximum(cnt_ref[...], 1.0)
    x = jnp.concatenate([
        h0_ref[...], h1_ref[...],
        t01_ref[0] * inv[:, 0:1], t20_ref[1] * inv[:, 0:1],
        t01_ref[1] * inv[:, 1:2], t12_ref[0] * inv[:, 1:2],
        t20_ref[0] * inv[:, 2:3], t12_ref[1] * inv[:, 2:3],
    ], axis=1)
    acc = lax.dot_general(x, w_ref[...], (((1,), (0,)), ((), ())),
                          preferred_element_type=jnp.float32)
    h2 = jnp.maximum(acc + b_ref[...], 0.0)

    mask = (lax.broadcasted_iota(jnp.int32, (_G, _BN), 0) == batch_ref[0]
            ).astype(jnp.float32)
    psum[...] += lax.dot_general(mask, h2, (((1,), (0,)), ((), ())),
                                 preferred_element_type=jnp.float32)
    gcnt[...] += jnp.broadcast_to(jnp.sum(mask, axis=1, keepdims=True),
                                  (_G, 128))

    @pl.when(i == _NBLK - 1)
    def _():
        pooled = psum[...] / jnp.maximum(gcnt[...][:, :1], 1.0)
        out_ref[...] = (lax.dot_general(
            pooled, lw_ref[...], (((1,), (0,)), ((), ())),
            preferred_element_type=jnp.float32) + lb_ref[...])


def _layer2_pool(h1c0, h1c1, t01, t20, t12, cntp, wc, b, batch3, lw, lb):
    return pl.pallas_call(
        _layer2_pool_body,
        grid=(_NBLK,),
        in_specs=[
            pl.BlockSpec((_BN, 128), lambda i: (i, 0)),
            pl.BlockSpec((_BN, 128), lambda i: (i, 0)),
            pl.BlockSpec((2, _BN, 128), lambda i: (0, i, 0)),
            pl.BlockSpec((2, _BN, 128), lambda i: (0, i, 0)),
            pl.BlockSpec((2, _BN, 128), lambda i: (0, i, 0)),
            pl.BlockSpec((_BN, 8), lambda i: (i, 0)),
            pl.BlockSpec((4 * _HID, _HID), lambda i: (0, 0)),
            pl.BlockSpec((1, _HID), lambda i: (0, 0)),
            pl.BlockSpec((1, 1, _BN), lambda i: (i, 0, 0)),
            pl.BlockSpec((_HID, _NCLS), lambda i: (0, 0)),
            pl.BlockSpec((1, _NCLS), lambda i: (0, 0)),
        ],
        out_specs=pl.BlockSpec((_G, _NCLS), lambda i: (0, 0)),
        out_shape=jax.ShapeDtypeStruct((_G, _NCLS), jnp.float32),
        scratch_shapes=[pltpu.VMEM((_G, _HID), jnp.float32),
                        pltpu.VMEM((_G, 128), jnp.float32)],
    )(h1c0, h1c1, t01, t20, t12, cntp, wc, b, batch3, lw, lb)


def kernel(x, edge_index, edge_type, batch, shape_emb, color_emb,
           W1, root1, b1, W2, root2, b2, lin_w, lin_b):
    f32 = jnp.float32
    s3 = x[:, 0].astype(jnp.int32).reshape(_NBLK, 1, _BN)
    c3 = x[:, 1].astype(jnp.int32).reshape(_NBLK, 1, _BN)
    batch3 = batch.astype(jnp.int32).reshape(_NBLK, 1, _BN)

    src = edge_index[0].astype(jnp.int32)
    dst = edge_index[1].astype(jnp.int32)
    rt = edge_type.astype(jnp.int32)
    dump = _N + jnp.arange(_NS, dtype=jnp.int32).reshape(_NS, 1, 1)

    src_f = src.reshape(_NS, _NB, _K)
    dst_f = dst.reshape(_NS, _NB, _K)
    rt_f = rt.reshape(_NS, _NB, _K)
    seg_f = [jnp.where(rt_f == r, dst_f, dump) for r in range(_R)]
    srcm_f = [jnp.where(rt_f == r, src_f, 0) for r in range(_R)]

    src_h = src.reshape(2, _NS, _NBH, _K)
    dst_h = dst.reshape(2, _NS, _NBH, _K)
    rt_h = rt.reshape(2, _NS, _NBH, _K)
    seg2_h = jnp.where(rt_h == 2, dst_h, dump)
    srcm2_h = jnp.where(rt_h == 2, src_h, 0)

    wc1 = jnp.concatenate([root1, W1[0], W1[1], W1[2], W1[2]], axis=0)
    wc2 = jnp.concatenate([root2, W2.reshape(_R * _HID, _HID)], axis=0)
    b1r = b1.reshape(1, _HID)
    b2r = b2.reshape(1, _HID)
    lbr = lin_b.reshape(1, _NCLS)

    za = jnp.zeros((_STR, 128), f32)

    gat_f = _make_sc_scatter(_NB)
    gat_h = _make_sc_scatter(_NBH)

    oh3 = jnp.eye(3, 128, dtype=f32)
    c_all = gat_h(oh3, oh3, dst_h[0], dst_h[1], rt_h[0], rt_h[1], za)
    cnt = c_all[0, :_N, :] + c_all[1, :_N, :]
    one = jnp.ones((_N,), f32)
    cntp = jnp.stack([cnt[:, 0], cnt[:, 1], cnt[:, 2], cnt[:, 2],
                      one, one, one, one], axis=1)

    h = _embed(s3, c3, shape_emb, color_emb)

    s01 = gat_f(h, h, seg_f[0], seg_f[1], srcm_f[0], srcm_f[1], za)
    s22 = gat_h(h, h, seg2_h[0], seg2_h[1], srcm2_h[0], srcm2_h[1], za)
    h1c0, h1c1 = _layer1(h, s01, s22, cntp, wc1, b1r)

    t01 = gat_f(h1c0, h1c0, seg_f[0], seg_f[1], srcm_f[0], srcm_f[1], za)
    t20 = gat_f(h1c0, h1c1, seg_f[2], seg_f[0], srcm_f[2], srcm_f[0], za)
    t12 = gat_f(h1c1, h1c1, seg_f[1], seg_f[2], srcm_f[1], srcm_f[2], za)
    return _layer2_pool(h1c0, h1c1, t01, t20, t12, cntp, wc2, b2r,
                        batch3, lin_w, lbr)


# speedup vs baseline: 48.5444x; 48.5444x over previous
import jax
import jax.numpy as jnp
from jax import lax
from jax.experimental import pallas as pl
from jax.experimental.pallas import tpu as pltpu
from jax.experimental.pallas import tpu_sc as plsc

_N = 10000
_E = 320000
_R = 3
_G = 64
_EMB = 128
_HID = 256
_NCLS = 10
_SLAB = 10112

_NS = 16
_K = 125
_NB = _E // _NS // _K
_NBH = _NB // 2
_CH = 40
_STR = _SLAB // _NS

_BN = 1000
_NBLK = _N // _BN


def _make_sc_scatter(gather: bool, nb: int):
    nc = nb // _CH
    mesh = plsc.VectorSubcoreMesh(core_axis_name="c", subcore_axis_name="s")
    out_type = jax.ShapeDtypeStruct((2, _SLAB, 128), jnp.float32)
    if gather:
        scratch = [
            pltpu.VMEM((_CH, _K), jnp.int32),
            pltpu.VMEM((_CH, _K), jnp.int32),
            pltpu.VMEM((2, _K, 128), jnp.float32),
            pltpu.VMEM_SHARED((_SLAB, 128), jnp.float32),
            pltpu.SemaphoreType.DMA,
            pltpu.SemaphoreType.DMA,
        ]
    else:
        scratch = [
            pltpu.VMEM((_CH, _K), jnp.int32),
            pltpu.VMEM((_K, 128), jnp.float32),
            pltpu.VMEM_SHARED((_SLAB, 128), jnp.float32),
        ]

    def body(*refs):
        if gather:
            (tbl0, tbl1, seg0_hbm, seg1_hbm, src0_hbm, src1_hbm, za,
             a_out, src_v, seg_v, rows_v, a_sh, sem0, sem1) = refs
        else:
            (ones_hbm, seg0_hbm, seg1_hbm, za,
             a_out, seg_v, rows_v, a_sh) = refs
        cid = lax.axis_index("c")
        sid = lax.axis_index("s")

        if not gather:
            pltpu.sync_copy(ones_hbm, rows_v)
        pltpu.sync_copy(za, a_sh.at[pl.ds(sid * _STR, _STR)])

        plsc.subcore_barrier()

        def edge_pass(tbl, seg_hbm, src_hbm):
            def chunk(ci, carry):
                pltpu.sync_copy(seg_hbm.at[sid, pl.ds(ci * _CH, _CH)], seg_v)
                if not gather:
                    def step(j, c2):
                        pltpu.sync_copy(rows_v, a_sh.at[seg_v.at[j]],
                                        add=True)
                        return c2
                    lax.fori_loop(0, _CH, step, 0)
                    return carry

                pltpu.sync_copy(src_hbm.at[sid, pl.ds(ci * _CH, _CH)], src_v)
                pltpu.async_copy(tbl.at[src_v.at[0]], rows_v.at[0], sem0)

                def pair(jp, c2):
                    j0 = 2 * jp
                    pltpu.make_async_copy(tbl.at[src_v.at[j0]],
                                          rows_v.at[0], sem0).wait()
                    pltpu.async_copy(tbl.at[src_v.at[j0 + 1]],
                                     rows_v.at[1], sem1)
                    pltpu.sync_copy(rows_v.at[0], a_sh.at[seg_v.at[j0]],
                                    add=True)
                    pltpu.make_async_copy(tbl.at[src_v.at[j0 + 1]],
                                          rows_v.at[1], sem1).wait()

                    @pl.when(jp < _CH // 2 - 1)
                    def _():
                        pltpu.async_copy(tbl.at[src_v.at[j0 + 2]],
                                         rows_v.at[0], sem0)

                    pltpu.sync_copy(rows_v.at[1], a_sh.at[seg_v.at[j0 + 1]],
                                    add=True)
                    return c2
                lax.fori_loop(0, _CH // 2, pair, 0)
                return carry
            lax.fori_loop(0, nc, chunk, 0)

        @pl.when(cid == 0)
        def _():
            edge_pass(tbl0 if gather else None, seg0_hbm,
                      src0_hbm if gather else None)

        @pl.when(cid == 1)
        def _():
            edge_pass(tbl1 if gather else None, seg1_hbm,
                      src1_hbm if gather else None)

        plsc.subcore_barrier()

        pltpu.sync_copy(a_sh.at[pl.ds(sid * _STR, _STR)],
                        a_out.at[cid, pl.ds(sid * _STR, _STR)])

    return pl.kernel(body, out_type=out_type, mesh=mesh,
                     scratch_types=scratch)


def _embed_body(s_ref, c_ref, se_ref, ce_ref, out_ref):
    sval = s_ref[0]
    cval = c_ref[0]
    ohs = (lax.broadcasted_iota(jnp.int32, (16, _BN), 0) == sval
           ).astype(jnp.float32)
    ohc = (lax.broadcasted_iota(jnp.int32, (16, _BN), 0) == cval
           ).astype(jnp.float32)
    dn = (((0,), (0,)), ((), ()))
    out_ref[...] = (
        lax.dot_general(ohs, se_ref[...], dn,
                        preferred_element_type=jnp.float32)
        + lax.dot_general(ohc, ce_ref[...], dn,
                          preferred_element_type=jnp.float32))


def _embed(s3, c3, se, ce):
    return pl.pallas_call(
        _embed_body,
        grid=(_NBLK,),
        in_specs=[
            pl.BlockSpec((1, 1, _BN), lambda i: (i, 0, 0)),
            pl.BlockSpec((1, 1, _BN), lambda i: (i, 0, 0)),
            pl.BlockSpec((16, _EMB), lambda i: (0, 0)),
            pl.BlockSpec((16, _EMB), lambda i: (0, 0)),
        ],
        out_specs=pl.BlockSpec((_BN, _EMB), lambda i: (i, 0)),
        out_shape=jax.ShapeDtypeStruct((_N, _EMB), jnp.float32),
    )(s3, c3, se, ce)


def _layer1_body(h_ref, s01_ref, s22_ref, cnt_ref, w_ref, b_ref,
                 out0_ref, out1_ref):
    inv = 1.0 / jnp.maximum(cnt_ref[...], 1.0)
    x = jnp.concatenate([
        h_ref[...],
        s01_ref[0] * inv[:, 0:1],
        s01_ref[1] * inv[:, 1:2],
        s22_ref[0] * inv[:, 2:3],
        s22_ref[1] * inv[:, 3:4],
    ], axis=1)
    acc = lax.dot_general(x, w_ref[...], (((1,), (0,)), ((), ())),
                          preferred_element_type=jnp.float32)
    acc = jnp.maximum(acc + b_ref[...], 0.0)
    out0_ref[...] = acc[:, :128]
    out1_ref[...] = acc[:, 128:]


def _layer1(h, s01, s22, cntp, wc, b):
    return pl.pallas_call(
        _layer1_body,
        grid=(_NBLK,),
        in_specs=[
            pl.BlockSpec((_BN, _EMB), lambda i: (i, 0)),
            pl.BlockSpec((2, _BN, 128), lambda i: (0, i, 0)),
            pl.BlockSpec((2, _BN, 128), lambda i: (0, i, 0)),
            pl.BlockSpec((_BN, 8), lambda i: (i, 0)),
            pl.BlockSpec((5 * _EMB, _HID), lambda i: (0, 0)),
            pl.BlockSpec((1, _HID), lambda i: (0, 0)),
        ],
        out_specs=[pl.BlockSpec((_BN, 128), lambda i: (i, 0)),
                   pl.BlockSpec((_BN, 128), lambda i: (i, 0))],
        out_shape=[jax.ShapeDtypeStruct((_N, 128), jnp.float32),
                   jax.ShapeDtypeStruct((_N, 128), jnp.float32)],
    )(h, s01, s22, cntp, wc, b)


def _layer2_pool_body(h0_ref, h1_ref, t01_ref, t20_ref, t12_ref, cnt_ref,
                      w_ref, b_ref, batch_ref, lw_ref, lb_ref, out_ref,
                      psum, gcnt):
    i = pl.program_id(0)

    @pl.when(i == 0)
    def _():
        psum[...] = jnp.zeros_like(psum)
        gcnt[...] = jnp.zeros_like(gcnt)

    inv = 1.0 / jnp.maximum(cnt_ref[...], 1.0)
    x = jnp.concatenate([
        h0_ref[...], h1_ref[...],
        t01_ref[0] * inv[:, 0:1], t20_ref[1] * inv[:, 0:1],
        t01_ref[1] * inv[:, 1:2], t12_ref[0] * inv[:, 1:2],
        t20_ref[0] * inv[:, 2:3], t12_ref[1] * inv[:, 2:3],
    ], axis=1)
    acc = lax.dot_general(x, w_ref[...], (((1,), (0,)), ((), ())),
                          preferred_element_type=jnp.float32)
    h2 = jnp.maximum(acc + b_ref[...], 0.0)

    mask = (lax.broadcasted_iota(jnp.int32, (_G, _BN), 0) == batch_ref[0]
            ).astype(jnp.float32)
    psum[...] += lax.dot_general(mask, h2, (((1,), (0,)), ((), ())),
                                 preferred_element_type=jnp.float32)
    gcnt[...] += jnp.broadcast_to(jnp.sum(mask, axis=1, keepdims=True),
                                  (_G, 128))

    @pl.when(i == _NBLK - 1)
    def _():
        pooled = psum[...] / jnp.maximum(gcnt[...][:, :1], 1.0)
        out_ref[...] = (lax.dot_general(
            pooled, lw_ref[...], (((1,), (0,)), ((), ())),
            preferred_element_type=jnp.float32) + lb_ref[...])


def _layer2_pool(h1c0, h1c1, t01, t20, t12, cntp, wc, b, batch3, lw, lb):
    return pl.pallas_call(
        _layer2_pool_body,
        grid=(_NBLK,),
        in_specs=[
            pl.BlockSpec((_BN, 128), lambda i: (i, 0)),
            pl.BlockSpec((_BN, 128), lambda i: (i, 0)),
            pl.BlockSpec((2, _BN, 128), lambda i: (0, i, 0)),
            pl.BlockSpec((2, _BN, 128), lambda i: (0, i, 0)),
            pl.BlockSpec((2, _BN, 128), lambda i: (0, i, 0)),
            pl.BlockSpec((_BN, 8), lambda i: (i, 0)),
            pl.BlockSpec((4 * _HID, _HID), lambda i: (0, 0)),
            pl.BlockSpec((1, _HID), lambda i: (0, 0)),
            pl.BlockSpec((1, 1, _BN), lambda i: (i, 0, 0)),
            pl.BlockSpec((_HID, _NCLS), lambda i: (0, 0)),
            pl.BlockSpec((1, _NCLS), lambda i: (0, 0)),
        ],
        out_specs=pl.BlockSpec((_G, _NCLS), lambda i: (0, 0)),
        out_shape=jax.ShapeDtypeStruct((_G, _NCLS), jnp.float32),
        scratch_shapes=[pltpu.VMEM((_G, _HID), jnp.float32),
                        pltpu.VMEM((_G, 128), jnp.float32)],
    )(h1c0, h1c1, t01, t20, t12, cntp, wc, b, batch3, lw, lb)


def kernel(x, edge_index, edge_type, batch, shape_emb, color_emb,
           W1, root1, b1, W2, root2, b2, lin_w, lin_b):
    f32 = jnp.float32
    s3 = x[:, 0].astype(jnp.int32).reshape(_NBLK, 1, _BN)
    c3 = x[:, 1].astype(jnp.int32).reshape(_NBLK, 1, _BN)
    batch3 = batch.astype(jnp.int32).reshape(_NBLK, 1, _BN)

    src = edge_index[0].astype(jnp.int32)
    dst = edge_index[1].astype(jnp.int32)
    rt = edge_type.astype(jnp.int32)
    dump = _N + jnp.arange(_NS, dtype=jnp.int32).reshape(_NS, 1, 1)

    src_f = src.reshape(_NS, _NB, _K)
    dst_f = dst.reshape(_NS, _NB, _K)
    rt_f = rt.reshape(_NS, _NB, _K)
    seg_f = [jnp.where(rt_f == r, dst_f, dump) for r in range(_R)]

    src_h = src.reshape(2, _NS, _NBH, _K)
    dst_h = dst.reshape(2, _NS, _NBH, _K)
    rt_h = rt.reshape(2, _NS, _NBH, _K)
    seg2_h = jnp.where(rt_h == 2, dst_h, dump)

    wc1 = jnp.concatenate([root1, W1[0], W1[1], W1[2], W1[2]], axis=0)
    wc2 = jnp.concatenate([root2, W2.reshape(_R * _HID, _HID)], axis=0)
    b1r = b1.reshape(1, _HID)
    b2r = b2.reshape(1, _HID)
    lbr = lin_b.reshape(1, _NCLS)

    za = jnp.zeros((_STR, 128), f32)
    ones_rows = jnp.ones((_K, 128), f32)

    gat_f = _make_sc_scatter(True, _NB)
    gat_h = _make_sc_scatter(True, _NBH)
    cnt_f = _make_sc_scatter(False, _NB)
    cnt_h = _make_sc_scatter(False, _NBH)

    c01 = cnt_f(ones_rows, seg_f[0], seg_f[1], za)
    c22 = cnt_h(ones_rows, seg2_h[0], seg2_h[1], za)
    cnt2 = c22[0, :_N, 0] + c22[1, :_N, 0]
    one = jnp.ones((_N,), f32)
    cntp = jnp.stack([c01[0, :_N, 0], c01[1, :_N, 0], cnt2, cnt2,
                      one, one, one, one], axis=1)

    h = _embed(s3, c3, shape_emb, color_emb)

    s01 = gat_f(h, h, seg_f[0], seg_f[1], src_f, src_f, za)
    s22 = gat_h(h, h, seg2_h[0], seg2_h[1], src_h[0], src_h[1], za)
    h1c0, h1c1 = _layer1(h, s01, s22, cntp, wc1, b1r)

    t01 = gat_f(h1c0, h1c0, seg_f[0], seg_f[1], src_f, src_f, za)
    t20 = gat_f(h1c0, h1c1, seg_f[2], seg_f[0], src_f, src_f, za)
    t12 = gat_f(h1c1, h1c1, seg_f[1], seg_f[2], src_f, src_f, za)
    return _layer2_pool(h1c0, h1c1, t01, t20, t12, cntp, wc2, b2r,
                        batch3, lin_w, lbr)
